# Initial kernel scaffold; baseline (speedup 1.0000x reference)
import functools

import jax
import jax.numpy as jnp
from jax import lax
from jax.experimental import pallas as pl
from jax.experimental.pallas import tpu as pltpu
from jax.experimental.pallas import tpu_sc as plsc

N = 10000
E = 160000
D = 300
DE = 16
L = 5
NL = 2000
NU = 256

DP = 320
DH = DP // 2
NSUB = 16
CH = 128
CPW = 80
E_PAD = NSUB * CPW * CH
N_ACC = 10016
BN = 400
BNP = 1000
BE = 2048


def _edge_proj_kernel(ea_ref, ew_ref, eb_ref, o_ref):
    v = jnp.dot(ea_ref[...], ew_ref[0], preferred_element_type=jnp.float32)
    v = v + eb_ref[...]
    o_ref[0, 0] = v[:, :DH]
    o_ref[0, 1] = v[:, DH:]


def _dense_kernel(a_ref, w_ref, b_ref, o_ref):
    a = jnp.concatenate([a_ref[0], a_ref[1]], axis=1)
    v = jnp.dot(a, w_ref[...], preferred_element_type=jnp.float32) + b_ref[...]
    v = jnp.maximum(v, 0.0)
    o_ref[0] = v[:, :DH]
    o_ref[1] = v[:, DH:]


def _lower_pool_kernel(lb_ref, h_ref, o_ref):
    i = pl.program_id(0)
    lb = lb_ref[0, 0]
    h = jnp.concatenate([h_ref[0], h_ref[1]], axis=1)
    haug = jnp.concatenate([h, jnp.ones((BNP, 8), jnp.float32)], axis=1)
    gi = lax.broadcasted_iota(jnp.int32, (NL, BNP), 0)
    ind = (gi == lb[None, :]).astype(jnp.float32)
    part = jnp.dot(ind, haug, preferred_element_type=jnp.float32)

    @pl.when(i == 0)
    def _():
        o_ref[...] = part

    @pl.when(i != 0)
    def _():
        o_ref[...] = o_ref[...] + part


def _final_kernel(p_ref, ub_ref, ub2_ref, c1_ref, c1b_ref, c2_ref, c2b_ref,
                  o_ref):
    pooled = p_ref[...]
    cnt = jnp.clip(pooled[:, DP:DP + 1], 1.0, None)
    lower = pooled[:, :DP] / cnt
    ub = ub_ref[0, 0]
    ub2 = ub2_ref[0, 0]
    gi = lax.broadcasted_iota(jnp.int32, (NU, NL), 0)
    uind = (gi == ub[None, :]).astype(jnp.float32)
    uind2 = (gi == ub2[None, :]).astype(jnp.float32)
    ucnt = jnp.clip(jnp.sum(uind, axis=1, keepdims=True), 1.0, None)
    out0 = jnp.dot(uind, lower, preferred_element_type=jnp.float32) / ucnt
    out1 = jnp.dot(uind2, lower, preferred_element_type=jnp.float32) / ucnt

    def classify(g):
        hc = jnp.dot(g, c1_ref[...], preferred_element_type=jnp.float32)
        hc = jnp.maximum(hc + c1b_ref[...], 0.0)
        return jnp.dot(hc, c2_ref[...],
                       preferred_element_type=jnp.float32) + c2b_ref[...]

    o_ref[...] = jnp.concatenate([classify(out0), classify(out1)], axis=0)


def _make_sc_layer(l):
    mesh = plsc.VectorSubcoreMesh(core_axis_name="c", subcore_axis_name="s")

    @functools.partial(
        pl.kernel,
        out_type=jax.ShapeDtypeStruct((2, N, DH), jnp.float32),
        scratch_types=[
            pltpu.VMEM((CPW, CH), jnp.int32),
            pltpu.VMEM((CPW, CH), jnp.int32),
            pltpu.VMEM((CH, DH), jnp.float32),
            pltpu.VMEM((CH, DH), jnp.float32),
            pltpu.VMEM_SHARED((N_ACC, DH), jnp.float32),
            pltpu.SemaphoreType.DMA,
        ],
        mesh=mesh,
    )
    def sc_layer(hflat, e_all, src2, dst3, zeros, out,
                 srcv, dstv, hbuf, ebuf, acc, sem):
        c = lax.axis_index("c")
        s = lax.axis_index("s")
        pltpu.sync_copy(zeros.at[pl.ds(s * 626, 626)],
                        acc.at[pl.ds(s * 626, 626)])
        pltpu.sync_copy(src2.at[c, s], srcv)
        pltpu.sync_copy(dst3.at[s], dstv)
        plsc.subcore_barrier()

        def chunk(j, carry):
            ebase = (s * CPW + j) * CH
            pltpu.sync_copy(e_all.at[l, c, pl.ds(ebase, CH)], ebuf)
            pltpu.async_copy(hflat.at[srcv.at[j]], hbuf, sem).wait()

            def row(r, carry2):
                for k in range(DH // 16):
                    sl = pl.ds(k * 16, 16)
                    hbuf[r, sl] = jnp.maximum(hbuf[r, sl] + ebuf[r, sl], 0.0)
                return carry2

            lax.fori_loop(0, CH, row, 0)
            pltpu.sync_copy(hbuf, acc.at[dstv.at[j]], add=True)
            return carry

        lax.fori_loop(0, CPW, chunk, 0)
        plsc.subcore_barrier()
        pltpu.sync_copy(acc.at[pl.ds(s * 625, 625)],
                        out.at[c, pl.ds(s * 625, 625)])

    return sc_layer


def kernel(x, edge_index, edge_attr, lower_batch, upper_batch,
           W, b, Ew, Eb, C1, c1b, C2, c2b):
    f32 = jnp.float32
    src = edge_index[0]
    dst = edge_index[1]
    pad = E_PAD - E
    src_p = jnp.concatenate([src, jnp.zeros((pad,), jnp.int32)])
    dst_p = jnp.concatenate([dst, jnp.full((pad,), N, jnp.int32)])
    ea_p = jnp.concatenate([edge_attr, jnp.zeros((pad, DE), f32)], axis=0)
    src3 = src_p.reshape(NSUB, CPW, CH)
    src2 = jnp.stack([src3, src3 + N])
    dst3 = dst_p.reshape(NSUB, CPW, CH)

    Ew_p = jnp.pad(Ew, ((0, 0), (0, 0), (0, DP - D)))
    Eb_p = jnp.pad(Eb, ((0, 0), (0, DP - D))).reshape(L, 1, DP)
    W_p = jnp.pad(W, ((0, 0), (0, DP - D), (0, DP - D)))
    b_p = jnp.pad(b, ((0, 0), (0, DP - D))).reshape(L, 1, DP)
    C1p = jnp.pad(C1, ((0, DP - D), (0, DP - D)))
    c1bp = jnp.pad(c1b, (0, DP - D)).reshape(1, DP)
    C2p = jnp.pad(C2, ((0, DP - D), (0, 127)))
    c2bp = jnp.pad(c2b, (0, 127)).reshape(1, 128)
    xp = jnp.pad(x, ((0, 0), (0, DP - D)))
    hflat = jnp.concatenate([xp[:, :DH], xp[:, DH:]], axis=0)
    zeros_acc = jnp.zeros((N_ACC, DH), f32)
    lb3 = lower_batch.reshape(N // BNP, 1, BNP)
    ub3 = upper_batch.reshape(1, 1, NL)
    ub23 = jnp.roll(upper_batch, -1).reshape(1, 1, NL)

    e_all = pl.pallas_call(
        _edge_proj_kernel,
        grid=(L, E_PAD // BE),
        in_specs=[
            pl.BlockSpec((BE, DE), lambda l_, i: (i, 0)),
            pl.BlockSpec((1, DE, DP), lambda l_, i: (l_, 0, 0)),
            pl.BlockSpec((1, DP), lambda l_, i: (l_, 0)),
        ],
        out_specs=pl.BlockSpec((1, 2, BE, DH), lambda l_, i: (l_, 0, i, 0)),
        out_shape=jax.ShapeDtypeStruct((L, 2, E_PAD, DH), f32),
    )(ea_p, Ew_p, Eb_p)

    dense = pl.pallas_call(
        _dense_kernel,
        grid=(N // BN,),
        in_specs=[
            pl.BlockSpec((2, BN, DH), lambda i: (0, i, 0)),
            pl.BlockSpec((DP, DP), lambda i: (0, 0)),
            pl.BlockSpec((1, DP), lambda i: (0, 0)),
        ],
        out_specs=pl.BlockSpec((2, BN, DH), lambda i: (0, i, 0)),
        out_shape=jax.ShapeDtypeStruct((2, N, DH), f32),
    )
    for l in range(L):
        agg2 = _make_sc_layer(l)(hflat, e_all, src2, dst3, zeros_acc)
        h2 = dense(agg2, W_p[l], b_p[l])
        hflat = h2.reshape(2 * N, DH)

    pooled = pl.pallas_call(
        _lower_pool_kernel,
        grid=(N // BNP,),
        in_specs=[
            pl.BlockSpec((1, 1, BNP), lambda i: (i, 0, 0)),
            pl.BlockSpec((2, BNP, DH), lambda i: (0, i, 0)),
        ],
        out_specs=pl.BlockSpec((NL, DP + 8), lambda i: (0, 0)),
        out_shape=jax.ShapeDtypeStruct((NL, DP + 8), f32),
    )(lb3, h2)

    fin = pl.pallas_call(
        _final_kernel,
        in_specs=[
            pl.BlockSpec((NL, DP + 8), lambda: (0, 0)),
            pl.BlockSpec((1, 1, NL), lambda: (0, 0, 0)),
            pl.BlockSpec((1, 1, NL), lambda: (0, 0, 0)),
            pl.BlockSpec((DP, DP), lambda: (0, 0)),
            pl.BlockSpec((1, DP), lambda: (0, 0)),
            pl.BlockSpec((DP, 128), lambda: (0, 0)),
            pl.BlockSpec((1, 128), lambda: (0, 0)),
        ],
        out_specs=pl.BlockSpec((2 * NU, 128), lambda: (0, 0)),
        out_shape=jax.ShapeDtypeStruct((2 * NU, 128), f32),
    )(pooled, ub3, ub23, C1p, c1bp, C2p, c2bp)

    logits = fin[:, 0]
    labels = jnp.concatenate([jnp.zeros((NU,), f32), jnp.ones((NU,), f32)])
    return logits, labels


# speedup vs baseline: 1.0569x; 1.0569x over previous
import functools

import jax
import jax.numpy as jnp
from jax import lax
from jax.experimental import pallas as pl
from jax.experimental.pallas import tpu as pltpu
from jax.experimental.pallas import tpu_sc as plsc

N = 10000
E = 160000
D = 300
DE = 16
L = 5
NL = 2000
NU = 256

DP = 320
DH = DP // 2
NSUB = 16
CH = 64
CPW = 160
E_PAD = NSUB * CPW * CH
N_ACC = 10016
N_OUT = N_ACC
BN = 400
BNP = 1000
BE = 2048


def _edge_proj_kernel(ea_ref, ew_ref, eb_ref, o_ref):
    v = jnp.dot(ea_ref[...], ew_ref[0], preferred_element_type=jnp.float32)
    v = v + eb_ref[0]
    o_ref[0, 0] = v[:, :DH]
    o_ref[0, 1] = v[:, DH:]


def _dense_kernel(a_ref, w_ref, b_ref, o_ref):
    a = jnp.concatenate([a_ref[0], a_ref[1]], axis=1)
    v = jnp.dot(a, w_ref[...], preferred_element_type=jnp.float32) + b_ref[...]
    v = jnp.maximum(v, 0.0)
    o_ref[0] = v[:, :DH]
    o_ref[1] = v[:, DH:]


def _lower_pool_kernel(lb_ref, h_ref, o_ref):
    i = pl.program_id(0)
    lb = lb_ref[0, 0]
    h = jnp.concatenate([h_ref[0], h_ref[1]], axis=1)
    haug = jnp.concatenate([h, jnp.ones((BNP, 8), jnp.float32)], axis=1)
    gi = lax.broadcasted_iota(jnp.int32, (NL, BNP), 0)
    ind = (gi == lb[None, :]).astype(jnp.float32)
    part = jnp.dot(ind, haug, preferred_element_type=jnp.float32)

    @pl.when(i == 0)
    def _():
        o_ref[...] = part

    @pl.when(i != 0)
    def _():
        o_ref[...] = o_ref[...] + part


def _final_kernel(p_ref, ub_ref, ub2_ref, c1_ref, c1b_ref, c2_ref, c2b_ref,
                  o_ref):
    pooled = p_ref[...]
    cnt = jnp.clip(pooled[:, DP:DP + 1], 1.0, None)
    lower = pooled[:, :DP] / cnt
    ub = ub_ref[0, 0]
    ub2 = ub2_ref[0, 0]
    gi = lax.broadcasted_iota(jnp.int32, (NU, NL), 0)
    uind = (gi == ub[None, :]).astype(jnp.float32)
    uind2 = (gi == ub2[None, :]).astype(jnp.float32)
    ucnt = jnp.clip(jnp.sum(uind, axis=1, keepdims=True), 1.0, None)
    out0 = jnp.dot(uind, lower, preferred_element_type=jnp.float32) / ucnt
    out1 = jnp.dot(uind2, lower, preferred_element_type=jnp.float32) / ucnt

    def classify(g):
        hc = jnp.dot(g, c1_ref[...], preferred_element_type=jnp.float32)
        hc = jnp.maximum(hc + c1b_ref[...], 0.0)
        return jnp.dot(hc, c2_ref[...],
                       preferred_element_type=jnp.float32) + c2b_ref[...]

    o_ref[...] = jnp.concatenate([classify(out0), classify(out1)], axis=0)


def _make_sc_layer(l):
    mesh = plsc.VectorSubcoreMesh(core_axis_name="c", subcore_axis_name="s")

    @functools.partial(
        pl.kernel,
        out_type=jax.ShapeDtypeStruct((2, N_OUT, DH), jnp.float32),
        scratch_types=[
            pltpu.VMEM((CPW, CH), jnp.int32),
            pltpu.VMEM((CH,), jnp.int32),
            pltpu.VMEM((CH, DH), jnp.float32),
            pltpu.VMEM((CH, DH), jnp.float32),
            pltpu.VMEM_SHARED((N_ACC, DH), jnp.float32),
            pltpu.SemaphoreType.DMA,
        ],
        mesh=mesh,
        compiler_params=pltpu.CompilerParams(use_tc_tiling_on_sc=False),
    )
    def sc_layer(hflat, e_all, src2, dst3, zeros, out,
                 srcv, dstc, hbuf, ebuf, acc, sem):
        c = lax.axis_index("c")
        s = lax.axis_index("s")
        pltpu.sync_copy(zeros.at[pl.ds(s * 626, 626)],
                        acc.at[pl.ds(s * 626, 626)])
        pltpu.sync_copy(src2.at[c, s], srcv)
        plsc.subcore_barrier()

        def chunk(j, carry):
            ebase = (s * CPW + j) * CH
            pltpu.sync_copy(dst3.at[s, j], dstc)
            pltpu.sync_copy(e_all.at[l, c, pl.ds(ebase, CH)], ebuf)
            pltpu.async_copy(hflat.at[srcv.at[j]], hbuf, sem).wait()

            def row(r, carry2):
                for k in range(DH // 16):
                    sl = pl.ds(k * 16, 16)
                    hbuf[r, sl] = jnp.maximum(hbuf[r, sl] + ebuf[r, sl], 0.0)
                return carry2

            lax.fori_loop(0, CH, row, 0)
            pltpu.sync_copy(hbuf, acc.at[dstc], add=True)
            return carry

        lax.fori_loop(0, CPW, chunk, 0)
        plsc.subcore_barrier()
        pltpu.sync_copy(acc.at[pl.ds(s * 626, 626)],
                        out.at[c, pl.ds(s * 626, 626)])

    return sc_layer


def kernel(x, edge_index, edge_attr, lower_batch, upper_batch,
           W, b, Ew, Eb, C1, c1b, C2, c2b):
    f32 = jnp.float32
    src = edge_index[0]
    dst = edge_index[1]
    pad = E_PAD - E
    src_p = jnp.concatenate([src, jnp.zeros((pad,), jnp.int32)])
    dst_p = jnp.concatenate([dst, jnp.full((pad,), N, jnp.int32)])
    ea_p = jnp.concatenate([edge_attr, jnp.zeros((pad, DE), f32)], axis=0)
    src3 = src_p.reshape(NSUB, CPW, CH)
    src2 = jnp.stack([src3, src3 + N])
    dst3 = dst_p.reshape(NSUB, CPW, CH)

    Ew_p = jnp.pad(Ew, ((0, 0), (0, 0), (0, DP - D)))
    Eb_p = jnp.pad(Eb, ((0, 0), (0, DP - D))).reshape(L, 1, DP)
    W_p = jnp.pad(W, ((0, 0), (0, DP - D), (0, DP - D)))
    b_p = jnp.pad(b, ((0, 0), (0, DP - D))).reshape(L, 1, DP)
    C1p = jnp.pad(C1, ((0, DP - D), (0, DP - D)))
    c1bp = jnp.pad(c1b, (0, DP - D)).reshape(1, DP)
    C2p = jnp.pad(C2, ((0, DP - D), (0, 127)))
    c2bp = jnp.pad(c2b, (0, 127)).reshape(1, 128)
    xp = jnp.pad(x, ((0, 0), (0, DP - D)))
    hflat = jnp.concatenate([xp[:, :DH], xp[:, DH:]], axis=0)
    zeros_acc = jnp.zeros((N_ACC, DH), f32)
    lb3 = lower_batch.reshape(N // BNP, 1, BNP)
    ub3 = upper_batch.reshape(1, 1, NL)
    ub23 = jnp.roll(upper_batch, -1).reshape(1, 1, NL)

    e_all = pl.pallas_call(
        _edge_proj_kernel,
        grid=(L, E_PAD // BE),
        in_specs=[
            pl.BlockSpec((BE, DE), lambda l_, i: (i, 0)),
            pl.BlockSpec((1, DE, DP), lambda l_, i: (l_, 0, 0)),
            pl.BlockSpec((1, 1, DP), lambda l_, i: (l_, 0, 0)),
        ],
        out_specs=pl.BlockSpec((1, 2, BE, DH), lambda l_, i: (l_, 0, i, 0)),
        out_shape=jax.ShapeDtypeStruct((L, 2, E_PAD, DH), f32),
    )(ea_p, Ew_p, Eb_p)

    dense = pl.pallas_call(
        _dense_kernel,
        grid=(N // BN,),
        in_specs=[
            pl.BlockSpec((2, BN, DH), lambda i: (0, i, 0)),
            pl.BlockSpec((DP, DP), lambda i: (0, 0)),
            pl.BlockSpec((1, DP), lambda i: (0, 0)),
        ],
        out_specs=pl.BlockSpec((2, BN, DH), lambda i: (0, i, 0)),
        out_shape=jax.ShapeDtypeStruct((2, N, DH), f32),
    )
    for l in range(L):
        agg2 = _make_sc_layer(l)(hflat, e_all, src2, dst3, zeros_acc)
        h2 = dense(agg2, W_p[l], b_p[l])
        hflat = h2.reshape(2 * N, DH)

    pooled = pl.pallas_call(
        _lower_pool_kernel,
        grid=(N // BNP,),
        in_specs=[
            pl.BlockSpec((1, 1, BNP), lambda i: (i, 0, 0)),
            pl.BlockSpec((2, BNP, DH), lambda i: (0, i, 0)),
        ],
        out_specs=pl.BlockSpec((NL, DP + 8), lambda i: (0, 0)),
        out_shape=jax.ShapeDtypeStruct((NL, DP + 8), f32),
    )(lb3, h2)

    fin = pl.pallas_call(
        _final_kernel,
        in_specs=[
            pl.BlockSpec((NL, DP + 8), lambda: (0, 0)),
            pl.BlockSpec((1, 1, NL), lambda: (0, 0, 0)),
            pl.BlockSpec((1, 1, NL), lambda: (0, 0, 0)),
            pl.BlockSpec((DP, DP), lambda: (0, 0)),
            pl.BlockSpec((1, DP), lambda: (0, 0)),
            pl.BlockSpec((DP, 128), lambda: (0, 0)),
            pl.BlockSpec((1, 128), lambda: (0, 0)),
        ],
        out_specs=pl.BlockSpec((2 * NU, 128), lambda: (0, 0)),
        out_shape=jax.ShapeDtypeStruct((2 * NU, 128), f32),
    )(pooled, ub3, ub23, C1p, c1bp, C2p, c2bp)

    logits = fin[:, 0]
    labels = jnp.concatenate([jnp.zeros((NU,), f32), jnp.ones((NU,), f32)])
    return logits, labels
